# baseline (speedup 1.0000x reference)
import functools

import jax
import jax.numpy as jnp
from jax import lax
from jax.experimental import pallas as pl
from jax.experimental.pallas import tpu as pltpu
from jax.experimental.pallas import tpu_sc as plsc

NC = 2
NS = 16
NW = NC * NS
GRP = 128


def _cdiv(a, b):
    return (a + b - 1) // b


_GDN = lax.GatherDimensionNumbers(
    offset_dims=(), collapsed_slice_dims=(0,), start_index_map=(0,))


def _bcast_lane(v, j):
    idx = jnp.full((16, 1), j, jnp.int32)
    return lax.gather(v, idx, _GDN, (1,),
                      mode=lax.GatherScatterMode.PROMISE_IN_BOUNDS)


def _tc1_body(x_ref, w1_ref, root1_ref, b1_ref, t1_ref, r1_ref):
    w1 = w1_ref[...]
    wc = jnp.concatenate([w1[0], w1[1] - w1[0], root1_ref[...]], axis=1)
    res = jnp.dot(x_ref[...], wc, preferred_element_type=jnp.float32)
    t1_ref[...] = res[:, :32]
    r1_ref[...] = res[:, 32:48] + b1_ref[...]


def _tc2_body(aggp_ref, cntp_ref, r1_ref, w2_ref, root2_ref, b2_ref,
              t2_ref, r2_ref):
    aggp = aggp_ref[...]
    cnt = jnp.maximum(cntp_ref[0] + cntp_ref[1], 1.0)
    v = (aggp[0] + aggp[1]) / cnt[:, None] + r1_ref[...]
    h = jnp.where(v > 0, v, jnp.exp(jnp.minimum(v, 0.0)) - 1.0)
    w2 = w2_ref[...]
    z1 = jnp.zeros((w2.shape[1], 1), jnp.float32)
    wf = jnp.concatenate([w2[0], z1, w2[1] - w2[0], z1, root2_ref[...]],
                         axis=1)
    res = jnp.dot(h, wf, preferred_element_type=jnp.float32)
    one7 = (lax.broadcasted_iota(jnp.int32, (1, 16), 1) == 7
            ).astype(jnp.float32)
    t2_ref[...] = res[:, :16] + one7
    r2_ref[...] = res[:, 16:23] + b2_ref[...]


def _tc3_body(agg2p_ref, cntp_ref, r2_ref, out_ref):
    a = agg2p_ref[0] + agg2p_ref[1]
    cnt = jnp.maximum(cntp_ref[0] + cntp_ref[1], 1.0)
    m7 = a[:, 0:7] + a[:, 8:15]
    z = m7 / cnt[:, None] + r2_ref[...]
    mx = jnp.max(z, axis=1, keepdims=True)
    s = jnp.log(jnp.sum(jnp.exp(z - mx), axis=1, keepdims=True))
    out_ref[...] = z - mx - s


DEPTH = 4


def _sc_edge_pass(table, ei3, fr3, *, n_pad, groups, layer1):
    rows_per_tile = n_pad // NS
    width = 32 if layer1 else 16
    mesh = plsc.VectorSubcoreMesh(core_axis_name="c", subcore_axis_name="s")

    if layer1:
        out_type = (jax.ShapeDtypeStruct((NC, n_pad, 16), jnp.float32),
                    jax.ShapeDtypeStruct((NC, n_pad), jnp.float32))
    else:
        out_type = jax.ShapeDtypeStruct((NC, n_pad, 16), jnp.float32)

    scratch = dict(
        sidx=pltpu.VMEM((groups, GRP), jnp.int32),
        didx=pltpu.VMEM((groups, GRP), jnp.int32),
        fv=pltpu.VMEM((groups, GRP), jnp.float32),
        zbuf=pltpu.VMEM((rows_per_tile, 16), jnp.float32),
        agg_sh=pltpu.VMEM_SHARED((n_pad, 16), jnp.float32),
        stsem=pltpu.SemaphoreType.DMA,
    )
    for b in range(DEPTH):
        scratch[f"rows{b}"] = pltpu.VMEM((GRP, width), jnp.float32)
        scratch[f"mv{b}"] = pltpu.VMEM((GRP, 16), jnp.float32)
    for b in range(DEPTH):
        scratch[f"gsem{b}"] = pltpu.SemaphoreType.DMA
        scratch[f"ssem{b}"] = pltpu.SemaphoreType.DMA
    if layer1:
        scratch["ones"] = pltpu.VMEM((GRP,), jnp.float32)
        scratch["zbuf1"] = pltpu.VMEM((rows_per_tile,), jnp.float32)
        scratch["cnt_sh"] = pltpu.VMEM_SHARED((n_pad,), jnp.float32)
        for b in range(DEPTH):
            scratch[f"csem{b}"] = pltpu.SemaphoreType.DMA

    @functools.partial(
        pl.kernel,
        out_type=out_type,
        mesh=mesh,
        compiler_params=pltpu.CompilerParams(use_tc_tiling_on_sc=False),
        scratch_types=scratch,
    )
    def k(t_hbm, ei_hbm, fr_hbm, *outs, **scr):
        if layer1:
            agg_out, cnt_out = outs
        else:
            agg_out, = outs
        sidx, didx, fv = scr["sidx"], scr["didx"], scr["fv"]
        zbuf, agg_sh, stsem = scr["zbuf"], scr["agg_sh"], scr["stsem"]
        rows = [scr[f"rows{b}"] for b in range(DEPTH)]
        mv = [scr[f"mv{b}"] for b in range(DEPTH)]
        gsem = [scr[f"gsem{b}"] for b in range(DEPTH)]
        ssem = [scr[f"ssem{b}"] for b in range(DEPTH)]
        if layer1:
            ones, zbuf1, cnt_sh = scr["ones"], scr["zbuf1"], scr["cnt_sh"]
            csem = [scr[f"csem{b}"] for b in range(DEPTH)]

        cid = lax.axis_index("c")
        sid = lax.axis_index("s")
        wid = sid * NC + cid
        gmax = groups - 1

        pltpu.async_copy(ei_hbm.at[0, wid], sidx, stsem)
        pltpu.async_copy(ei_hbm.at[1, wid], didx, stsem)
        pltpu.async_copy(fr_hbm.at[wid], fv, stsem)

        z16 = jnp.zeros((16,), jnp.float32)

        def zrow(i, _):
            zbuf[i, :] = z16
            for b in range(DEPTH):
                mv[b][i % GRP, :] = z16
            return _
        lax.fori_loop(0, rows_per_tile, zrow, None)
        base = sid * rows_per_tile
        pltpu.sync_copy(zbuf, agg_sh.at[pl.ds(base, rows_per_tile)])
        if layer1:
            for i in range(rows_per_tile // 16):
                zbuf1[pl.ds(i * 16, 16)] = z16
            pltpu.sync_copy(zbuf1, cnt_sh.at[pl.ds(base, rows_per_tile)])
            for i in range(GRP // 16):
                ones[pl.ds(i * 16, 16)] = jnp.ones((16,), jnp.float32)
        pltpu.make_async_copy(ei_hbm.at[0, wid], sidx, stsem).wait()
        pltpu.make_async_copy(ei_hbm.at[1, wid], didx, stsem).wait()
        pltpu.make_async_copy(fr_hbm.at[wid], fv, stsem).wait()
        plsc.subcore_barrier()

        lmask = lax.iota(jnp.int32, 16) < 8

        for b in range(DEPTH):
            pltpu.async_copy(mv[b], agg_sh.at[didx.at[b]], ssem[b],
                             add=True)
            pltpu.async_copy(t_hbm.at[sidx.at[b]], rows[b], gsem[b])
            if layer1:
                pltpu.async_copy(zbuf1.at[pl.ds(0, GRP)],
                                 cnt_sh.at[didx.at[b]], csem[b], add=True)

        def wait_gather(b):
            pltpu.make_async_copy(t_hbm.at[pl.ds(0, GRP)], rows[b],
                                  gsem[b]).wait()

        def wait_scatter(b):
            pltpu.make_async_copy(agg_out.at[0, pl.ds(0, GRP)], mv[b],
                                  ssem[b]).wait()

        def wait_cnt(b):
            pltpu.make_async_copy(fr_hbm.at[0, 0], ones, csem[b]).wait()

        def do_group(g, b):
            wait_gather(b)
            wait_scatter(b)
            if layer1:
                wait_cnt(b)
            for j in range(GRP // 16):
                fvreg = fv[g, pl.ds(j * 16, 16)]
                for jj in range(16):
                    e = j * 16 + jj
                    fb = _bcast_lane(fvreg, jj)
                    if layer1:
                        a = rows[b][e, pl.ds(0, 16)]
                        bb = rows[b][e, pl.ds(16, 16)]
                        mv[b][e, :] = a + fb * bb
                    else:
                        r = rows[b][e, :]
                        mv[b][e, :] = r * jnp.where(lmask, 1.0, fb)
            pltpu.async_copy(mv[b], agg_sh.at[didx.at[g]], ssem[b],
                             add=True)
            if layer1:
                pltpu.async_copy(ones, cnt_sh.at[didx.at[g]], csem[b],
                                 add=True)
            gnext = jnp.minimum(g + DEPTH, gmax)
            pltpu.async_copy(t_hbm.at[sidx.at[gnext]], rows[b], gsem[b])

        def body(i, _):
            for b in range(DEPTH):
                do_group(DEPTH * i + b, b)
            return _
        lax.fori_loop(0, groups // DEPTH, body, None)
        for b in range(DEPTH):
            wait_gather(b)
            wait_scatter(b)
            if layer1:
                wait_cnt(b)
        plsc.subcore_barrier()

        pltpu.sync_copy(agg_sh.at[pl.ds(base, rows_per_tile)],
                        agg_out.at[cid, pl.ds(base, rows_per_tile)])
        if layer1:
            pltpu.sync_copy(cnt_sh.at[pl.ds(base, rows_per_tile)],
                            cnt_out.at[cid, pl.ds(base, rows_per_tile)])

    return k(table, ei3, fr3)


def kernel(x, edge_index, edge_attr, W1, root1, b1, W2, root2, b2):
    n, f_in = x.shape
    e = edge_index.shape[1]
    h_dim = W1.shape[2]
    c_dim = W2.shape[2]

    groups = DEPTH * _cdiv(e, NW * DEPTH * GRP)
    e_pad = NW * groups * GRP
    n_pad = _cdiv(n + 16, NS * 16) * NS * 16

    pad = e_pad - e
    ar = jnp.arange(pad, dtype=jnp.int32)
    padei = jnp.stack([ar % n, n + (ar % 16)])
    ei3 = jnp.concatenate([edge_index, padei], axis=1
                          ).reshape(2, NW, groups, GRP)
    fr3 = jnp.concatenate([edge_attr,
                           jnp.zeros((pad, 1), jnp.float32)], axis=0
                          ).reshape(NW, groups, GRP)

    rb1 = 2000
    g1 = n // rb1
    t1, r1 = pl.pallas_call(
        _tc1_body,
        grid=(g1,),
        in_specs=[
            pl.BlockSpec((rb1, f_in), lambda i: (i, 0)),
            pl.BlockSpec((2, f_in, h_dim), lambda i: (0, 0, 0)),
            pl.BlockSpec((f_in, h_dim), lambda i: (0, 0)),
            pl.BlockSpec((1, h_dim), lambda i: (0, 0)),
        ],
        out_specs=[
            pl.BlockSpec((rb1, 2 * h_dim), lambda i: (i, 0)),
            pl.BlockSpec((rb1, h_dim), lambda i: (i, 0)),
        ],
        out_shape=[
            jax.ShapeDtypeStruct((n_pad, 2 * h_dim), jnp.float32),
            jax.ShapeDtypeStruct((n_pad, h_dim), jnp.float32),
        ],
    )(x, W1, root1, b1.reshape(1, h_dim))

    aggp, cntp = _sc_edge_pass(t1, ei3, fr3, n_pad=n_pad,
                               groups=groups, layer1=True)

    rb = 1024
    gx = n_pad // rb
    t2, r2 = pl.pallas_call(
        _tc2_body,
        grid=(gx,),
        in_specs=[
            pl.BlockSpec((2, rb, h_dim), lambda i: (0, i, 0)),
            pl.BlockSpec((2, rb), lambda i: (0, i)),
            pl.BlockSpec((rb, h_dim), lambda i: (i, 0)),
            pl.BlockSpec((2, h_dim, c_dim), lambda i: (0, 0, 0)),
            pl.BlockSpec((h_dim, c_dim), lambda i: (0, 0)),
            pl.BlockSpec((1, c_dim), lambda i: (0, 0)),
        ],
        out_specs=[
            pl.BlockSpec((rb, 16), lambda i: (i, 0)),
            pl.BlockSpec((rb, c_dim), lambda i: (i, 0)),
        ],
        out_shape=[
            jax.ShapeDtypeStruct((n_pad, 16), jnp.float32),
            jax.ShapeDtypeStruct((n_pad, c_dim), jnp.float32),
        ],
    )(aggp, cntp, r1, W2, root2, b2.reshape(1, c_dim))

    agg2p = _sc_edge_pass(t2, ei3, fr3, n_pad=n_pad,
                          groups=groups, layer1=False)

    out = pl.pallas_call(
        _tc3_body,
        grid=(gx,),
        in_specs=[
            pl.BlockSpec((2, rb, 16), lambda i: (0, i, 0)),
            pl.BlockSpec((2, rb), lambda i: (0, i)),
            pl.BlockSpec((rb, c_dim), lambda i: (i, 0)),
        ],
        out_specs=pl.BlockSpec((rb, c_dim), lambda i: (i, 0)),
        out_shape=jax.ShapeDtypeStruct((n, c_dim), jnp.float32),
    )(agg2p, cntp, r2)

    return out


# speedup vs baseline: 1.0457x; 1.0457x over previous
import functools

import jax
import jax.numpy as jnp
from jax import lax
from jax.experimental import pallas as pl
from jax.experimental.pallas import tpu as pltpu
from jax.experimental.pallas import tpu_sc as plsc

NC = 2
NS = 16
NW = NC * NS
GRP = 128


def _cdiv(a, b):
    return (a + b - 1) // b


_GDN = lax.GatherDimensionNumbers(
    offset_dims=(), collapsed_slice_dims=(0,), start_index_map=(0,))


def _bcast_lane(v, j):
    idx = jnp.full((16, 1), j, jnp.int32)
    return lax.gather(v, idx, _GDN, (1,),
                      mode=lax.GatherScatterMode.PROMISE_IN_BOUNDS)


def _tc1_body(x_ref, w1_ref, root1_ref, b1_ref, t1_ref, r1_ref):
    w1 = w1_ref[...]
    wc = jnp.concatenate([w1[0], w1[1] - w1[0], root1_ref[...]], axis=1)
    res = jnp.dot(x_ref[...], wc, preferred_element_type=jnp.float32)
    t1_ref[...] = res[:, :32]
    r1_ref[...] = res[:, 32:48] + b1_ref[...]


def _tc2_body(aggp_ref, cntp_ref, r1_ref, w2_ref, root2_ref, b2_ref,
              t2_ref, r2_ref):
    aggp = aggp_ref[...]
    cnt = jnp.maximum(cntp_ref[0] + cntp_ref[1], 1.0)
    v = (aggp[0] + aggp[1]) / cnt[:, None] + r1_ref[...]
    h = jnp.where(v > 0, v, jnp.exp(jnp.minimum(v, 0.0)) - 1.0)
    w2 = w2_ref[...]
    z1 = jnp.zeros((w2.shape[1], 1), jnp.float32)
    wf = jnp.concatenate([w2[0], z1, w2[1] - w2[0], z1, root2_ref[...]],
                         axis=1)
    res = jnp.dot(h, wf, preferred_element_type=jnp.float32)
    one7 = (lax.broadcasted_iota(jnp.int32, (1, 16), 1) == 7
            ).astype(jnp.float32)
    t2_ref[...] = res[:, :16] + one7
    r2_ref[...] = res[:, 16:23] + b2_ref[...]


def _tc3_body(agg2p_ref, cntp_ref, r2_ref, out_ref):
    a = agg2p_ref[0] + agg2p_ref[1]
    cnt = jnp.maximum(cntp_ref[0] + cntp_ref[1], 1.0)
    m7 = a[:, 0:7] + a[:, 8:15]
    z = m7 / cnt[:, None] + r2_ref[...]
    mx = jnp.max(z, axis=1, keepdims=True)
    s = jnp.log(jnp.sum(jnp.exp(z - mx), axis=1, keepdims=True))
    out_ref[...] = z - mx - s


DEPTH = 4


def _sc_edge_pass(table, ei3, fr3, *, n_pad, groups, layer1):
    rows_per_tile = n_pad // NS
    width = 32 if layer1 else 16
    mesh = plsc.VectorSubcoreMesh(core_axis_name="c", subcore_axis_name="s")

    if layer1:
        out_type = (jax.ShapeDtypeStruct((NC, n_pad, 16), jnp.float32),
                    jax.ShapeDtypeStruct((NC, n_pad), jnp.float32))
    else:
        out_type = jax.ShapeDtypeStruct((NC, n_pad, 16), jnp.float32)

    scratch = dict(
        sidx=pltpu.VMEM((groups, GRP), jnp.int32),
        didx=pltpu.VMEM((groups, GRP), jnp.int32),
        fv=pltpu.VMEM((groups, GRP), jnp.float32),
        zbuf=pltpu.VMEM((rows_per_tile, 16), jnp.float32),
        agg_sh=pltpu.VMEM_SHARED((n_pad, 16), jnp.float32),
        stsem=pltpu.SemaphoreType.DMA,
    )
    for b in range(DEPTH):
        scratch[f"rows{b}"] = pltpu.VMEM((GRP, width), jnp.float32)
        scratch[f"mv{b}"] = pltpu.VMEM((GRP, 16), jnp.float32)
    for b in range(DEPTH):
        scratch[f"gsem{b}"] = pltpu.SemaphoreType.DMA
        scratch[f"ssem{b}"] = pltpu.SemaphoreType.DMA
    if layer1:
        scratch["ones"] = pltpu.VMEM((GRP,), jnp.float32)
        scratch["zbuf1"] = pltpu.VMEM((rows_per_tile,), jnp.float32)
        scratch["cnt_sh"] = pltpu.VMEM_SHARED((n_pad,), jnp.float32)
        for b in range(DEPTH):
            scratch[f"csem{b}"] = pltpu.SemaphoreType.DMA

    @functools.partial(
        pl.kernel,
        out_type=out_type,
        mesh=mesh,
        compiler_params=pltpu.CompilerParams(use_tc_tiling_on_sc=False),
        scratch_types=scratch,
    )
    def k(t_hbm, ei_hbm, fr_hbm, *outs, **scr):
        if layer1:
            agg_out, cnt_out = outs
        else:
            agg_out, = outs
        sidx, didx, fv = scr["sidx"], scr["didx"], scr["fv"]
        zbuf, agg_sh, stsem = scr["zbuf"], scr["agg_sh"], scr["stsem"]
        rows = [scr[f"rows{b}"] for b in range(DEPTH)]
        mv = [scr[f"mv{b}"] for b in range(DEPTH)]
        gsem = [scr[f"gsem{b}"] for b in range(DEPTH)]
        ssem = [scr[f"ssem{b}"] for b in range(DEPTH)]
        if layer1:
            ones, zbuf1, cnt_sh = scr["ones"], scr["zbuf1"], scr["cnt_sh"]
            csem = [scr[f"csem{b}"] for b in range(DEPTH)]

        cid = lax.axis_index("c")
        sid = lax.axis_index("s")
        wid = sid * NC + cid
        gmax = groups - 1

        pltpu.async_copy(ei_hbm.at[0, wid], sidx, stsem)
        pltpu.async_copy(ei_hbm.at[1, wid], didx, stsem)
        pltpu.async_copy(fr_hbm.at[wid], fv, stsem)

        z16 = jnp.zeros((16,), jnp.float32)

        def zrow(i, _):
            zbuf[i, :] = z16
            for b in range(DEPTH):
                mv[b][i % GRP, :] = z16
            return _
        lax.fori_loop(0, rows_per_tile, zrow, None)
        base = sid * rows_per_tile
        pltpu.sync_copy(zbuf, agg_sh.at[pl.ds(base, rows_per_tile)])
        if layer1:
            for i in range(rows_per_tile // 16):
                zbuf1[pl.ds(i * 16, 16)] = z16
            pltpu.sync_copy(zbuf1, cnt_sh.at[pl.ds(base, rows_per_tile)])
            for i in range(GRP // 16):
                ones[pl.ds(i * 16, 16)] = jnp.ones((16,), jnp.float32)
        pltpu.make_async_copy(ei_hbm.at[0, wid], sidx, stsem).wait()
        pltpu.make_async_copy(ei_hbm.at[1, wid], didx, stsem).wait()
        pltpu.make_async_copy(fr_hbm.at[wid], fv, stsem).wait()
        plsc.subcore_barrier()

        lmask = lax.iota(jnp.int32, 16) < 8

        for b in range(DEPTH):
            pltpu.async_copy(mv[b], agg_sh.at[didx.at[b]], ssem[b],
                             add=True)
            pltpu.async_copy(t_hbm.at[sidx.at[b]], rows[b], gsem[b])
            if layer1:
                pltpu.async_copy(zbuf1.at[pl.ds(0, GRP)],
                                 cnt_sh.at[didx.at[b]], csem[b], add=True)

        def wait_gather(b):
            pltpu.make_async_copy(t_hbm.at[pl.ds(0, GRP)], rows[b],
                                  gsem[b]).wait()

        def wait_scatter(b):
            pltpu.make_async_copy(agg_out.at[0, pl.ds(0, GRP)], mv[b],
                                  ssem[b]).wait()

        def wait_cnt(b):
            pltpu.make_async_copy(fr_hbm.at[0, 0], ones, csem[b]).wait()

        def do_group(g, b):
            wait_gather(b)
            wait_scatter(b)
            if layer1:
                wait_cnt(b)
            for j in range(GRP // 16):
                fvreg = fv[g, pl.ds(j * 16, 16)]
                for jj in range(16):
                    e = j * 16 + jj
                    fb = _bcast_lane(fvreg, jj)
                    if layer1:
                        a = rows[b][e, pl.ds(0, 16)]
                        bb = rows[b][e, pl.ds(16, 16)]
                        mv[b][e, :] = a + fb * bb
                    else:
                        r = rows[b][e, :]
                        mv[b][e, :] = r * jnp.where(lmask, 1.0, fb)
            pltpu.async_copy(mv[b], agg_sh.at[didx.at[g]], ssem[b],
                             add=True)
            if layer1:
                pltpu.async_copy(ones, cnt_sh.at[didx.at[g]], csem[b],
                                 add=True)
            gnext = jnp.minimum(g + DEPTH, gmax)
            pltpu.async_copy(t_hbm.at[sidx.at[gnext]], rows[b], gsem[b])

        def body(i, _):
            for b in range(DEPTH):
                do_group(DEPTH * i + b, b)
            return _
        lax.fori_loop(0, groups // DEPTH, body, None)
        for b in range(DEPTH):
            wait_gather(b)
            wait_scatter(b)
            if layer1:
                wait_cnt(b)
        plsc.subcore_barrier()

        pltpu.sync_copy(agg_sh.at[pl.ds(base, rows_per_tile)],
                        agg_out.at[cid, pl.ds(base, rows_per_tile)])
        if layer1:
            pltpu.sync_copy(cnt_sh.at[pl.ds(base, rows_per_tile)],
                            cnt_out.at[cid, pl.ds(base, rows_per_tile)])

    return k(table, ei3, fr3)


def kernel(x, edge_index, edge_attr, W1, root1, b1, W2, root2, b2):
    n, f_in = x.shape
    e = edge_index.shape[1]
    h_dim = W1.shape[2]
    c_dim = W2.shape[2]

    groups = DEPTH * _cdiv(e, NW * DEPTH * GRP)
    e_pad = NW * groups * GRP
    n_pad = _cdiv(n + 16, NS * 16) * NS * 16

    pad = e_pad - e
    ar = jnp.arange(pad, dtype=jnp.int32)
    padei = jnp.stack([ar % n, n + (ar % 16)])
    ei3 = jnp.concatenate([edge_index, padei], axis=1
                          ).reshape(2, NW, groups, GRP)
    fr3 = jnp.concatenate([edge_attr,
                           jnp.zeros((pad, 1), jnp.float32)], axis=0
                          ).reshape(NW, groups, GRP)

    rb1 = 2000
    g1 = n // rb1
    t1, r1 = pl.pallas_call(
        _tc1_body,
        grid=(g1,),
        in_specs=[
            pl.BlockSpec((rb1, f_in), lambda i: (i, 0)),
            pl.BlockSpec((2, f_in, h_dim), lambda i: (0, 0, 0)),
            pl.BlockSpec((f_in, h_dim), lambda i: (0, 0)),
            pl.BlockSpec((1, h_dim), lambda i: (0, 0)),
        ],
        out_specs=[
            pl.BlockSpec((rb1, 2 * h_dim), lambda i: (i, 0)),
            pl.BlockSpec((rb1, h_dim), lambda i: (i, 0)),
        ],
        out_shape=[
            jax.ShapeDtypeStruct((n_pad, 2 * h_dim), jnp.float32),
            jax.ShapeDtypeStruct((n_pad, h_dim), jnp.float32),
        ],
    )(x, W1, root1, b1.reshape(1, h_dim))

    aggp, cntp = _sc_edge_pass(t1, ei3, fr3, n_pad=n_pad,
                               groups=groups, layer1=True)

    rb = 5120
    gx = n_pad // rb
    t2, r2 = pl.pallas_call(
        _tc2_body,
        grid=(gx,),
        in_specs=[
            pl.BlockSpec((2, rb, h_dim), lambda i: (0, i, 0)),
            pl.BlockSpec((2, rb), lambda i: (0, i)),
            pl.BlockSpec((rb, h_dim), lambda i: (i, 0)),
            pl.BlockSpec((2, h_dim, c_dim), lambda i: (0, 0, 0)),
            pl.BlockSpec((h_dim, c_dim), lambda i: (0, 0)),
            pl.BlockSpec((1, c_dim), lambda i: (0, 0)),
        ],
        out_specs=[
            pl.BlockSpec((rb, 16), lambda i: (i, 0)),
            pl.BlockSpec((rb, c_dim), lambda i: (i, 0)),
        ],
        out_shape=[
            jax.ShapeDtypeStruct((n_pad, 16), jnp.float32),
            jax.ShapeDtypeStruct((n_pad, c_dim), jnp.float32),
        ],
    )(aggp, cntp, r1, W2, root2, b2.reshape(1, c_dim))

    agg2p = _sc_edge_pass(t2, ei3, fr3, n_pad=n_pad,
                          groups=groups, layer1=False)

    out = pl.pallas_call(
        _tc3_body,
        grid=(gx,),
        in_specs=[
            pl.BlockSpec((2, rb, 16), lambda i: (0, i, 0)),
            pl.BlockSpec((2, rb), lambda i: (0, i)),
            pl.BlockSpec((rb, c_dim), lambda i: (i, 0)),
        ],
        out_specs=pl.BlockSpec((rb, c_dim), lambda i: (i, 0)),
        out_shape=jax.ShapeDtypeStruct((n, c_dim), jnp.float32),
    )(agg2p, cntp, r2)

    return out


# speedup vs baseline: 1.0605x; 1.0141x over previous
import functools

import jax
import jax.numpy as jnp
from jax import lax
from jax.experimental import pallas as pl
from jax.experimental.pallas import tpu as pltpu
from jax.experimental.pallas import tpu_sc as plsc

NC = 2
NS = 16
NW = NC * NS
GRP = 128


def _cdiv(a, b):
    return (a + b - 1) // b


_GDN = lax.GatherDimensionNumbers(
    offset_dims=(), collapsed_slice_dims=(0,), start_index_map=(0,))


def _bcast_lane(v, j):
    idx = jnp.full((16, 1), j, jnp.int32)
    return lax.gather(v, idx, _GDN, (1,),
                      mode=lax.GatherScatterMode.PROMISE_IN_BOUNDS)


def _tc1_body(x_ref, w1_ref, root1_ref, b1_ref, t1_ref, r1_ref):
    w1 = w1_ref[...]
    wc = jnp.concatenate([w1[0], w1[1] - w1[0], root1_ref[...]], axis=1)
    res = jnp.dot(x_ref[...], wc, preferred_element_type=jnp.float32)
    t1_ref[...] = res[:, :32]
    r1_ref[...] = res[:, 32:48] + b1_ref[...]


def _tc2_body(aggp_ref, cntp_ref, r1_ref, w2_ref, root2_ref, b2_ref,
              t2_ref, r2_ref):
    aggp = aggp_ref[...]
    cnt = jnp.maximum(cntp_ref[0] + cntp_ref[1], 1.0)
    v = (aggp[0] + aggp[1]) / cnt[:, None] + r1_ref[...]
    h = jnp.where(v > 0, v, jnp.exp(jnp.minimum(v, 0.0)) - 1.0)
    w2 = w2_ref[...]
    z1 = jnp.zeros((w2.shape[1], 1), jnp.float32)
    wf = jnp.concatenate([w2[0], z1, w2[1] - w2[0], z1, root2_ref[...]],
                         axis=1)
    res = jnp.dot(h, wf, preferred_element_type=jnp.float32)
    one7 = (lax.broadcasted_iota(jnp.int32, (1, 16), 1) == 7
            ).astype(jnp.float32)
    t2_ref[...] = res[:, :16] + one7
    r2_ref[...] = res[:, 16:23] + b2_ref[...]


def _tc3_body(agg2p_ref, cntp_ref, r2_ref, out_ref):
    a = agg2p_ref[0] + agg2p_ref[1]
    cnt = jnp.maximum(cntp_ref[0] + cntp_ref[1], 1.0)
    m7 = a[:, 0:7] + a[:, 8:15]
    z = m7 / cnt[:, None] + r2_ref[...]
    mx = jnp.max(z, axis=1, keepdims=True)
    s = jnp.log(jnp.sum(jnp.exp(z - mx), axis=1, keepdims=True))
    out_ref[...] = z - mx - s


DEPTH = 4


def _sc_edge_pass(table, ei3, fr3, *, n_pad, groups, layer1):
    rows_per_tile = n_pad // NS
    width = 32 if layer1 else 16
    mesh = plsc.VectorSubcoreMesh(core_axis_name="c", subcore_axis_name="s")

    if layer1:
        out_type = (jax.ShapeDtypeStruct((NC, n_pad, 16), jnp.float32),
                    jax.ShapeDtypeStruct((NC, n_pad), jnp.float32))
    else:
        out_type = jax.ShapeDtypeStruct((NC, n_pad, 16), jnp.float32)

    scratch = dict(
        sidx=pltpu.VMEM((groups, GRP), jnp.int32),
        didx=pltpu.VMEM((groups, GRP), jnp.int32),
        fv=pltpu.VMEM((groups, GRP), jnp.float32),
        zbuf=pltpu.VMEM((rows_per_tile, 16), jnp.float32),
        agg_sh=pltpu.VMEM_SHARED((n_pad, 16), jnp.float32),
        stsem=pltpu.SemaphoreType.DMA,
    )
    for b in range(DEPTH):
        scratch[f"rows{b}"] = pltpu.VMEM((GRP, width), jnp.float32)
        scratch[f"mv{b}"] = pltpu.VMEM((GRP, 16), jnp.float32)
    for b in range(DEPTH):
        scratch[f"gsem{b}"] = pltpu.SemaphoreType.DMA
        scratch[f"ssem{b}"] = pltpu.SemaphoreType.DMA
    if layer1:
        scratch["ones"] = pltpu.VMEM((GRP,), jnp.float32)
        scratch["zbuf1"] = pltpu.VMEM((rows_per_tile,), jnp.float32)
        scratch["cnt_sh"] = pltpu.VMEM_SHARED((n_pad,), jnp.float32)
        for b in range(DEPTH):
            scratch[f"csem{b}"] = pltpu.SemaphoreType.DMA

    @functools.partial(
        pl.kernel,
        out_type=out_type,
        mesh=mesh,
        compiler_params=pltpu.CompilerParams(use_tc_tiling_on_sc=False),
        scratch_types=scratch,
    )
    def k(t_hbm, ei_hbm, fr_hbm, *outs, **scr):
        if layer1:
            agg_out, cnt_out = outs
        else:
            agg_out, = outs
        sidx, didx, fv = scr["sidx"], scr["didx"], scr["fv"]
        zbuf, agg_sh, stsem = scr["zbuf"], scr["agg_sh"], scr["stsem"]
        rows = [scr[f"rows{b}"] for b in range(DEPTH)]
        mv = [scr[f"mv{b}"] for b in range(DEPTH)]
        gsem = [scr[f"gsem{b}"] for b in range(DEPTH)]
        ssem = [scr[f"ssem{b}"] for b in range(DEPTH)]
        if layer1:
            ones, zbuf1, cnt_sh = scr["ones"], scr["zbuf1"], scr["cnt_sh"]
            csem = [scr[f"csem{b}"] for b in range(DEPTH)]

        cid = lax.axis_index("c")
        sid = lax.axis_index("s")
        wid = sid * NC + cid
        gmax = groups - 1

        pltpu.async_copy(ei_hbm.at[0, wid], sidx, stsem)
        pltpu.async_copy(ei_hbm.at[1, wid], didx, stsem)
        pltpu.async_copy(fr_hbm.at[wid], fv, stsem)

        z16 = jnp.zeros((16,), jnp.float32)

        def zrow(i, _):
            zbuf[i, :] = z16
            for b in range(DEPTH):
                mv[b][i % GRP, :] = z16
            return _
        lax.fori_loop(0, rows_per_tile, zrow, None)
        base = sid * rows_per_tile
        pltpu.sync_copy(zbuf, agg_sh.at[pl.ds(base, rows_per_tile)])
        if layer1:
            for i in range(rows_per_tile // 16):
                zbuf1[pl.ds(i * 16, 16)] = z16
            pltpu.sync_copy(zbuf1, cnt_sh.at[pl.ds(base, rows_per_tile)])
            for i in range(GRP // 16):
                ones[pl.ds(i * 16, 16)] = jnp.ones((16,), jnp.float32)
        pltpu.make_async_copy(ei_hbm.at[0, wid], sidx, stsem).wait()
        pltpu.make_async_copy(ei_hbm.at[1, wid], didx, stsem).wait()
        pltpu.make_async_copy(fr_hbm.at[wid], fv, stsem).wait()
        plsc.subcore_barrier()

        lmask = lax.iota(jnp.int32, 16) < 8

        for b in range(DEPTH):
            pltpu.async_copy(mv[b], agg_sh.at[didx.at[b]], ssem[b],
                             add=True)
            pltpu.async_copy(t_hbm.at[sidx.at[b]], rows[b], gsem[b])
            if layer1:
                pltpu.async_copy(zbuf1.at[pl.ds(0, GRP)],
                                 cnt_sh.at[didx.at[b]], csem[b], add=True)

        def wait_gather(b):
            pltpu.make_async_copy(t_hbm.at[pl.ds(0, GRP)], rows[b],
                                  gsem[b]).wait()

        def wait_scatter(b):
            pltpu.make_async_copy(agg_out.at[0, pl.ds(0, GRP)], mv[b],
                                  ssem[b]).wait()

        def wait_cnt(b):
            pltpu.make_async_copy(fr_hbm.at[0, 0], ones, csem[b]).wait()

        def do_group(g, b):
            wait_gather(b)
            wait_scatter(b)
            if layer1:
                wait_cnt(b)
            for j in range(GRP // 16):
                fvreg = fv[g, pl.ds(j * 16, 16)]
                for jj in range(16):
                    e = j * 16 + jj
                    fb = _bcast_lane(fvreg, jj)
                    if layer1:
                        a = rows[b][e, pl.ds(0, 16)]
                        bb = rows[b][e, pl.ds(16, 16)]
                        mv[b][e, :] = a + fb * bb
                    else:
                        r = rows[b][e, :]
                        mv[b][e, :] = r * jnp.where(lmask, 1.0, fb)
            pltpu.async_copy(mv[b], agg_sh.at[didx.at[g]], ssem[b],
                             add=True)
            if layer1:
                pltpu.async_copy(ones, cnt_sh.at[didx.at[g]], csem[b],
                                 add=True)
            gnext = jnp.minimum(g + DEPTH, gmax)
            pltpu.async_copy(t_hbm.at[sidx.at[gnext]], rows[b], gsem[b])

        def body(i, _):
            for b in range(DEPTH):
                do_group(DEPTH * i + b, b)
            return _
        lax.fori_loop(0, groups // DEPTH, body, None)
        for b in range(DEPTH):
            wait_gather(b)
            wait_scatter(b)
            if layer1:
                wait_cnt(b)
        plsc.subcore_barrier()

        pltpu.sync_copy(agg_sh.at[pl.ds(base, rows_per_tile)],
                        agg_out.at[cid, pl.ds(base, rows_per_tile)])
        if layer1:
            pltpu.sync_copy(cnt_sh.at[pl.ds(base, rows_per_tile)],
                            cnt_out.at[cid, pl.ds(base, rows_per_tile)])

    return k(table, ei3, fr3)


def kernel(x, edge_index, edge_attr, W1, root1, b1, W2, root2, b2):
    n, f_in = x.shape
    e = edge_index.shape[1]
    h_dim = W1.shape[2]
    c_dim = W2.shape[2]

    groups = DEPTH * _cdiv(e, NW * DEPTH * GRP)
    e_pad = NW * groups * GRP
    n_pad = _cdiv(n + 16, NS * 16) * NS * 16

    pad = e_pad - e
    ar = jnp.arange(pad, dtype=jnp.int32)
    padei = jnp.stack([ar % n, n + (ar % 16)])
    ei3 = jnp.concatenate([edge_index, padei], axis=1
                          ).reshape(2, NW, groups, GRP)
    fr3 = jnp.concatenate([edge_attr,
                           jnp.zeros((pad, 1), jnp.float32)], axis=0
                          ).reshape(NW, groups, GRP)

    rb1 = 5000
    g1 = n // rb1
    t1, r1 = pl.pallas_call(
        _tc1_body,
        grid=(g1,),
        in_specs=[
            pl.BlockSpec((rb1, f_in), lambda i: (i, 0)),
            pl.BlockSpec((2, f_in, h_dim), lambda i: (0, 0, 0)),
            pl.BlockSpec((f_in, h_dim), lambda i: (0, 0)),
            pl.BlockSpec((1, h_dim), lambda i: (0, 0)),
        ],
        out_specs=[
            pl.BlockSpec((rb1, 2 * h_dim), lambda i: (i, 0)),
            pl.BlockSpec((rb1, h_dim), lambda i: (i, 0)),
        ],
        out_shape=[
            jax.ShapeDtypeStruct((n_pad, 2 * h_dim), jnp.float32),
            jax.ShapeDtypeStruct((n_pad, h_dim), jnp.float32),
        ],
    )(x, W1, root1, b1.reshape(1, h_dim))

    aggp, cntp = _sc_edge_pass(t1, ei3, fr3, n_pad=n_pad,
                               groups=groups, layer1=True)

    rb = 5120
    gx = n_pad // rb
    t2, r2 = pl.pallas_call(
        _tc2_body,
        grid=(gx,),
        in_specs=[
            pl.BlockSpec((2, rb, h_dim), lambda i: (0, i, 0)),
            pl.BlockSpec((2, rb), lambda i: (0, i)),
            pl.BlockSpec((rb, h_dim), lambda i: (i, 0)),
            pl.BlockSpec((2, h_dim, c_dim), lambda i: (0, 0, 0)),
            pl.BlockSpec((h_dim, c_dim), lambda i: (0, 0)),
            pl.BlockSpec((1, c_dim), lambda i: (0, 0)),
        ],
        out_specs=[
            pl.BlockSpec((rb, 16), lambda i: (i, 0)),
            pl.BlockSpec((rb, c_dim), lambda i: (i, 0)),
        ],
        out_shape=[
            jax.ShapeDtypeStruct((n_pad, 16), jnp.float32),
            jax.ShapeDtypeStruct((n_pad, c_dim), jnp.float32),
        ],
    )(aggp, cntp, r1, W2, root2, b2.reshape(1, c_dim))

    agg2p = _sc_edge_pass(t2, ei3, fr3, n_pad=n_pad,
                          groups=groups, layer1=False)

    out = pl.pallas_call(
        _tc3_body,
        grid=(gx,),
        in_specs=[
            pl.BlockSpec((2, rb, 16), lambda i: (0, i, 0)),
            pl.BlockSpec((2, rb), lambda i: (0, i)),
            pl.BlockSpec((rb, c_dim), lambda i: (i, 0)),
        ],
        out_specs=pl.BlockSpec((rb, c_dim), lambda i: (i, 0)),
        out_shape=jax.ShapeDtypeStruct((n, c_dim), jnp.float32),
    )(agg2p, cntp, r2)

    return out
